# Initial kernel scaffold; baseline (speedup 1.0000x reference)
import jax
import jax.numpy as jnp
from jax.experimental import pallas as pl


def kernel(x, edge_index, W_in, b_in, W_gat, att_src, att_dst, b_gat, W_ih, W_hh, W_out, b_out):
    raise NotImplementedError("write your pallas kernel here")


# speedup vs baseline: 23.9727x; 23.9727x over previous
import functools

import jax
import jax.numpy as jnp
from jax import lax
from jax.experimental import pallas as pl
from jax.experimental.pallas import tpu as pltpu
from jax.experimental.pallas import tpu_sc as plsc

_N = 10000
_E = 320000
_H = 128
_NPAD = 10240
_NC, _NS = 2, 16
_T = _E // (_NC * _NS)
_C = 80
_DPT = _NPAD // _NS


def _mm_bias_body(x_ref, w_ref, b_ref, o_ref):
    o_ref[...] = jnp.dot(x_ref[...], w_ref[...],
                         preferred_element_type=jnp.float32) + b_ref[...]


def _mm_bias(x, w, b, blk=2000):
    n, _ = x.shape
    ko = w.shape[1]
    return pl.pallas_call(
        _mm_bias_body,
        grid=(n // blk,),
        in_specs=[pl.BlockSpec((blk, x.shape[1]), lambda i: (i, 0)),
                  pl.BlockSpec(w.shape, lambda i: (0, 0)),
                  pl.BlockSpec((1, ko), lambda i: (0, 0))],
        out_specs=pl.BlockSpec((blk, ko), lambda i: (i, 0)),
        out_shape=jax.ShapeDtypeStruct((n, ko), jnp.float32),
    )(x, w, b.reshape(1, -1))


def _stage_a_body(cur_ref, w_ref, av_ref, h_ref, a2_ref):
    h = jnp.dot(cur_ref[...], w_ref[...], preferred_element_type=jnp.float32)
    h_ref[...] = h
    a2_ref[...] = jnp.dot(h, av_ref[...], preferred_element_type=jnp.float32)


def _stage_a(cur, w, av, blk=2000):
    return pl.pallas_call(
        _stage_a_body,
        grid=(_N // blk,),
        in_specs=[pl.BlockSpec((blk, _H), lambda i: (i, 0)),
                  pl.BlockSpec((_H, _H), lambda i: (0, 0)),
                  pl.BlockSpec((_H, 8), lambda i: (0, 0))],
        out_specs=[pl.BlockSpec((blk, _H), lambda i: (i, 0)),
                   pl.BlockSpec((blk, 8), lambda i: (i, 0))],
        out_shape=[jax.ShapeDtypeStruct((_N, _H), jnp.float32),
                   jax.ShapeDtypeStruct((_N, 8), jnp.float32)],
    )(cur, w, av)


def _stage_c_body(acc_ref, den_ref, bg_ref, wih_ref, whh_ref, h_ref, c_ref,
                  ho_ref, co_ref):
    den = den_ref[0] + den_ref[1]
    acc = acc_ref[0] + acc_ref[1]
    bx = jnp.tanh(acc / (den + 1e-16) + bg_ref[...])
    gates = (jnp.dot(bx, wih_ref[...], preferred_element_type=jnp.float32)
             + jnp.dot(h_ref[...], whh_ref[...],
                       preferred_element_type=jnp.float32))
    gi = gates[:, 0 * _H:1 * _H]
    gf = gates[:, 1 * _H:2 * _H]
    gg = gates[:, 2 * _H:3 * _H]
    go = gates[:, 3 * _H:4 * _H]
    c_new = jax.nn.sigmoid(gf) * c_ref[...] + jax.nn.sigmoid(gi) * jnp.tanh(gg)
    co_ref[...] = c_new
    ho_ref[...] = jax.nn.sigmoid(go) * jnp.tanh(c_new)


def _stage_c(acc, den, bg, wihT, whhT, h_st, c_st, blk=2000):
    return pl.pallas_call(
        _stage_c_body,
        grid=(_N // blk,),
        in_specs=[pl.BlockSpec((2, blk, _H), lambda i: (0, i, 0)),
                  pl.BlockSpec((2, blk, 1), lambda i: (0, i, 0)),
                  pl.BlockSpec((1, _H), lambda i: (0, 0)),
                  pl.BlockSpec((_H, 4 * _H), lambda i: (0, 0)),
                  pl.BlockSpec((_H, 4 * _H), lambda i: (0, 0)),
                  pl.BlockSpec((blk, _H), lambda i: (i, 0)),
                  pl.BlockSpec((blk, _H), lambda i: (i, 0))],
        out_specs=[pl.BlockSpec((blk, _H), lambda i: (i, 0)),
                   pl.BlockSpec((blk, _H), lambda i: (i, 0))],
        out_shape=[jax.ShapeDtypeStruct((_N, _H), jnp.float32),
                   jax.ShapeDtypeStruct((_N, _H), jnp.float32)],
    )(acc, den, bg.reshape(1, -1), wihT, whhT, h_st, c_st)


def _build_edge_kernel():
    mesh = plsc.VectorSubcoreMesh(core_axis_name="c", subcore_axis_name="s")

    @functools.partial(
        pl.kernel,
        out_type=[jax.ShapeDtypeStruct((_NC, _NPAD, _H), jnp.float32),
                  jax.ShapeDtypeStruct((_NC, _NPAD), jnp.float32)],
        mesh=mesh,
        compiler_params=pltpu.CompilerParams(needs_layout_passes=False),
        scratch_types=[
            pltpu.VMEM((_N,), jnp.float32),
            pltpu.VMEM((_N,), jnp.float32),
            pltpu.VMEM((_C,), jnp.int32),
            pltpu.VMEM((_C,), jnp.int32),
            pltpu.VMEM((_C,), jnp.float32),
            pltpu.VMEM((_C, _H), jnp.float32),
            pltpu.VMEM((_DPT,), jnp.float32),
            pltpu.VMEM_SHARED((_NPAD, _H), jnp.float32),
            pltpu.VMEM_SHARED((_NPAD,), jnp.float32),
            pltpu.SemaphoreType.DMA,
        ],
    )
    def edge_kernel(h_hbm, asrc_hbm, adst_hbm, src_hbm, dst_hbm,
                    acc_hbm, den_hbm,
                    asrc_l, adst_l, sidx, didx, w_l, rows, zbuf,
                    acc_sh, den_sh, sem):
        cid = lax.axis_index("c")
        sid = lax.axis_index("s")
        wid = cid * _NS + sid

        pltpu.sync_copy(asrc_hbm, asrc_l)
        pltpu.sync_copy(adst_hbm, adst_l)

        def mx_body(i, carry):
            ms, md = carry
            return (jnp.maximum(ms, asrc_l[pl.ds(i * 16, 16)]),
                    jnp.maximum(md, adst_l[pl.ds(i * 16, 16)]))
        neg = jnp.full((16,), -3.0e38, jnp.float32)
        ms, md = lax.fori_loop(0, _N // 16, mx_body, (neg, neg))
        lanes = lax.iota(jnp.int32, 16)
        for sft in (8, 4, 2, 1):
            perm = jnp.bitwise_xor(lanes, sft)
            w_l[pl.ds(0, 16)] = ms
            ms = jnp.maximum(ms, plsc.load_gather(w_l, [perm]))
            w_l[pl.ds(0, 16)] = md
            md = jnp.maximum(md, plsc.load_gather(w_l, [perm]))
        m = ms + md
        big_m = jnp.where(m >= 0.0, m, 0.2 * m)

        zrow = jnp.zeros((16,), jnp.float32)

        def zr_body(j, _):
            for g in range(_H // 16):
                rows[j, pl.ds(g * 16, 16)] = zrow
            return 0
        lax.fori_loop(0, _C, zr_body, 0)

        def zd_body(j, _):
            zbuf[pl.ds(j * 16, 16)] = zrow
            return 0
        lax.fori_loop(0, _DPT // 16, zd_body, 0)

        rbase = pl.multiple_of(sid * _DPT, 8)
        for z in range(_DPT // _C):
            pltpu.sync_copy(rows, acc_sh.at[pl.ds(rbase + z * _C, _C)])
        pltpu.sync_copy(zbuf, den_sh.at[pl.ds(rbase, _DPT)])
        plsc.subcore_barrier()

        def chunk_body(k, _):
            ebase = wid * _T + k * _C
            pltpu.sync_copy(src_hbm.at[pl.ds(ebase, _C)], sidx)
            pltpu.sync_copy(dst_hbm.at[pl.ds(ebase, _C)], didx)
            cp = pltpu.async_copy(h_hbm.at[sidx], rows, sem)
            for j in range(_C // 16):
                sv = sidx[pl.ds(j * 16, 16)]
                dv = didx[pl.ds(j * 16, 16)]
                e = (plsc.load_gather(asrc_l, [sv])
                     + plsc.load_gather(adst_l, [dv]))
                e = jnp.where(e >= 0.0, e, 0.2 * e)
                w_l[pl.ds(j * 16, 16)] = jnp.exp(e - big_m)
            pltpu.sync_copy(w_l, den_sh.at[didx], add=True)
            cp.wait()

            def scale_body(t16, _):
                wv16 = w_l[pl.ds(t16 * 16, 16)]
                for lane in range(16):
                    t = t16 * 16 + lane
                    wv = lax.broadcast(wv16[lane], (16,))
                    for g in range(_H // 16):
                        rows[t, pl.ds(g * 16, 16)] = (
                            rows[t, pl.ds(g * 16, 16)] * wv)
                return 0
            lax.fori_loop(0, _C // 16, scale_body, 0)
            pltpu.sync_copy(rows, acc_sh.at[didx], add=True)
            return 0
        lax.fori_loop(0, _T // _C, chunk_body, 0)

        plsc.subcore_barrier()
        pltpu.sync_copy(acc_sh.at[pl.ds(rbase, _DPT)],
                        acc_hbm.at[cid, pl.ds(rbase, _DPT)])
        pltpu.sync_copy(den_sh.at[pl.ds(rbase, _DPT)],
                        den_hbm.at[cid, pl.ds(rbase, _DPT)])

    return edge_kernel


_edge_call = _build_edge_kernel()


def kernel(x, edge_index, W_in, b_in, W_gat, att_src, att_dst, b_gat,
           W_ih, W_hh, W_out, b_out):
    src = edge_index[0]
    dst = edge_index[1]
    cur = _mm_bias(x, W_in, b_in)
    h_st = jnp.zeros((_N, _H), jnp.float32)
    c_st = jnp.zeros((_N, _H), jnp.float32)
    hops = W_gat.shape[0]
    for i in range(hops):
        av = jnp.zeros((_H, 8), jnp.float32)
        av = av.at[:, 0].set(att_src[i]).at[:, 1].set(att_dst[i])
        h, a2 = _stage_a(cur, W_gat[i], av)
        acc, den = _edge_call(h, a2[:, 0], a2[:, 1], src, dst)
        h_st, c_st = _stage_c(acc, den[:, :, None], b_gat[i],
                              W_ih[i].T, W_hh[i].T, h_st, c_st)
        cur = h_st
    return _mm_bias(cur, W_out, b_out)


# speedup vs baseline: 33.0041x; 1.3767x over previous
import functools

import jax
import jax.numpy as jnp
from jax import lax
from jax.experimental import pallas as pl
from jax.experimental.pallas import tpu as pltpu
from jax.experimental.pallas import tpu_sc as plsc

_N = 10000
_E = 320000
_H = 128
_NPAD = 10240
_NC, _NS = 2, 16
_T = _E // (_NC * _NS)
_C = 80
_DPT = _NPAD // _NS


def _mm_bias_body(x_ref, w_ref, b_ref, o_ref):
    o_ref[...] = jnp.dot(x_ref[...], w_ref[...],
                         preferred_element_type=jnp.float32) + b_ref[...]


def _mm_bias(x, w, b, blk=2000):
    n, _ = x.shape
    ko = w.shape[1]
    return pl.pallas_call(
        _mm_bias_body,
        grid=(n // blk,),
        in_specs=[pl.BlockSpec((blk, x.shape[1]), lambda i: (i, 0)),
                  pl.BlockSpec(w.shape, lambda i: (0, 0)),
                  pl.BlockSpec((1, ko), lambda i: (0, 0))],
        out_specs=pl.BlockSpec((blk, ko), lambda i: (i, 0)),
        out_shape=jax.ShapeDtypeStruct((n, ko), jnp.float32),
    )(x, w, b.reshape(1, -1))


def _stage_a_body(cur_ref, w_ref, av_ref, h_ref, a2_ref):
    h = jnp.dot(cur_ref[...], w_ref[...], preferred_element_type=jnp.float32)
    h_ref[...] = h
    a2_ref[...] = jnp.dot(h, av_ref[...], preferred_element_type=jnp.float32)


def _stage_a(cur, w, av, blk=2000):
    return pl.pallas_call(
        _stage_a_body,
        grid=(_N // blk,),
        in_specs=[pl.BlockSpec((blk, _H), lambda i: (i, 0)),
                  pl.BlockSpec((_H, _H), lambda i: (0, 0)),
                  pl.BlockSpec((_H, 8), lambda i: (0, 0))],
        out_specs=[pl.BlockSpec((blk, _H), lambda i: (i, 0)),
                   pl.BlockSpec((blk, 8), lambda i: (i, 0))],
        out_shape=[jax.ShapeDtypeStruct((_N, _H), jnp.float32),
                   jax.ShapeDtypeStruct((_N, 8), jnp.float32)],
    )(cur, w, av)


def _stage_c_body(acc_ref, den_ref, bg_ref, wih_ref, whh_ref, h_ref, c_ref,
                  ho_ref, co_ref):
    den = den_ref[0] + den_ref[1]
    acc = acc_ref[0] + acc_ref[1]
    bx = jnp.tanh(acc / (den + 1e-16) + bg_ref[...])
    gates = (jnp.dot(bx, wih_ref[...], preferred_element_type=jnp.float32)
             + jnp.dot(h_ref[...], whh_ref[...],
                       preferred_element_type=jnp.float32))
    gi = gates[:, 0 * _H:1 * _H]
    gf = gates[:, 1 * _H:2 * _H]
    gg = gates[:, 2 * _H:3 * _H]
    go = gates[:, 3 * _H:4 * _H]
    c_new = jax.nn.sigmoid(gf) * c_ref[...] + jax.nn.sigmoid(gi) * jnp.tanh(gg)
    co_ref[...] = c_new
    ho_ref[...] = jax.nn.sigmoid(go) * jnp.tanh(c_new)


def _stage_c(acc, den, bg, wihT, whhT, h_st, c_st, blk=2000):
    return pl.pallas_call(
        _stage_c_body,
        grid=(_N // blk,),
        in_specs=[pl.BlockSpec((2, blk, _H), lambda i: (0, i, 0)),
                  pl.BlockSpec((2, blk, 1), lambda i: (0, i, 0)),
                  pl.BlockSpec((1, _H), lambda i: (0, 0)),
                  pl.BlockSpec((_H, 4 * _H), lambda i: (0, 0)),
                  pl.BlockSpec((_H, 4 * _H), lambda i: (0, 0)),
                  pl.BlockSpec((blk, _H), lambda i: (i, 0)),
                  pl.BlockSpec((blk, _H), lambda i: (i, 0))],
        out_specs=[pl.BlockSpec((blk, _H), lambda i: (i, 0)),
                   pl.BlockSpec((blk, _H), lambda i: (i, 0))],
        out_shape=[jax.ShapeDtypeStruct((_N, _H), jnp.float32),
                   jax.ShapeDtypeStruct((_N, _H), jnp.float32)],
    )(acc, den, bg.reshape(1, -1), wihT, whhT, h_st, c_st)


def _build_edge_kernel():
    mesh = plsc.VectorSubcoreMesh(core_axis_name="c", subcore_axis_name="s")

    @functools.partial(
        pl.kernel,
        out_type=[jax.ShapeDtypeStruct((_NC, _NPAD, _H), jnp.float32),
                  jax.ShapeDtypeStruct((_NC, _NPAD), jnp.float32)],
        mesh=mesh,
        compiler_params=pltpu.CompilerParams(needs_layout_passes=False),
        scratch_types=[
            pltpu.VMEM((_N,), jnp.float32),
            pltpu.VMEM((_N,), jnp.float32),
            pltpu.VMEM((2, _C), jnp.int32),
            pltpu.VMEM((2, _C), jnp.int32),
            pltpu.VMEM((_C,), jnp.float32),
            pltpu.VMEM((_C,), jnp.float32),
            pltpu.VMEM((_C, _H), jnp.float32),
            pltpu.VMEM((_C, _H), jnp.float32),
            pltpu.VMEM((_DPT,), jnp.float32),
            pltpu.VMEM_SHARED((_NPAD, _H), jnp.float32),
            pltpu.VMEM_SHARED((_NPAD,), jnp.float32),
            pltpu.SemaphoreType.DMA,
            pltpu.SemaphoreType.DMA,
            pltpu.SemaphoreType.DMA,
            pltpu.SemaphoreType.DMA,
            pltpu.SemaphoreType.DMA,
            pltpu.SemaphoreType.DMA,
        ],
    )
    def edge_kernel(h_hbm, asrc_hbm, adst_hbm, src_hbm, dst_hbm,
                    acc_hbm, den_hbm,
                    asrc_l, adst_l, ei0, ei1, w0, w1, r0, r1, zbuf,
                    acc_sh, den_sh, sga0, sga1, ssd0, ssd1, ssc0, ssc1):
        cid = lax.axis_index("c")
        sid = lax.axis_index("s")
        wid = cid * _NS + sid
        eidx = (ei0, ei1)
        wbuf = (w0, w1)
        rows = (r0, r1)
        sga = (sga0, sga1)
        ssd = (ssd0, ssd1)
        ssc = (ssc0, ssc1)
        nch = _T // _C

        pltpu.sync_copy(asrc_hbm, asrc_l)
        pltpu.sync_copy(adst_hbm, adst_l)

        def mx_body(i, carry):
            ms, md = carry
            return (jnp.maximum(ms, asrc_l[pl.ds(i * 16, 16)]),
                    jnp.maximum(md, adst_l[pl.ds(i * 16, 16)]))
        neg = jnp.full((16,), -3.0e38, jnp.float32)
        ms, md = lax.fori_loop(0, _N // 16, mx_body, (neg, neg))
        lanes = lax.iota(jnp.int32, 16)
        for sft in (8, 4, 2, 1):
            perm = jnp.bitwise_xor(lanes, sft)
            w0[pl.ds(0, 16)] = ms
            ms = jnp.maximum(ms, plsc.load_gather(w0, [perm]))
            w0[pl.ds(0, 16)] = md
            md = jnp.maximum(md, plsc.load_gather(w0, [perm]))
        m = ms + md
        big_m = jnp.where(m >= 0.0, m, 0.2 * m)

        zrow = jnp.zeros((16,), jnp.float32)

        def zr_body(j, _):
            for g in range(_H // 16):
                r0[j, pl.ds(g * 16, 16)] = zrow
            return 0
        lax.fori_loop(0, _C, zr_body, 0)

        def zd_body(j, _):
            zbuf[pl.ds(j * 16, 16)] = zrow
            return 0
        lax.fori_loop(0, _DPT // 16, zd_body, 0)

        rbase = pl.multiple_of(sid * _DPT, 8)
        for z in range(_DPT // _C):
            pltpu.sync_copy(r0, acc_sh.at[pl.ds(rbase + z * _C, _C)])
        pltpu.sync_copy(zbuf, den_sh.at[pl.ds(rbase, _DPT)])
        plsc.subcore_barrier()

        ebase0 = wid * _T

        def do_chunk(k, b, skip_wait, prefetch):
            o = 1 - b
            if prefetch:
                if not skip_wait:
                    pltpu.make_async_copy(
                        rows[o], acc_sh.at[eidx[o].at[1]], ssc[o]).wait()
                    pltpu.make_async_copy(
                        wbuf[o], den_sh.at[eidx[o].at[1]], ssd[o]).wait()
                nb = ebase0 + (k + 1) * _C
                pltpu.sync_copy(src_hbm.at[pl.ds(nb, _C)], eidx[o].at[0])
                pltpu.sync_copy(dst_hbm.at[pl.ds(nb, _C)], eidx[o].at[1])
                pltpu.async_copy(h_hbm.at[eidx[o].at[0]], rows[o], sga[o])
            for j in range(_C // 16):
                sv = eidx[b][0, pl.ds(j * 16, 16)]
                dv = eidx[b][1, pl.ds(j * 16, 16)]
                e = (plsc.load_gather(asrc_l, [sv])
                     + plsc.load_gather(adst_l, [dv]))
                e = jnp.where(e >= 0.0, e, 0.2 * e)
                wbuf[b][pl.ds(j * 16, 16)] = jnp.exp(e - big_m)
            pltpu.async_copy(wbuf[b], den_sh.at[eidx[b].at[1]], ssd[b],
                             add=True)
            pltpu.make_async_copy(h_hbm.at[eidx[b].at[0]], rows[b],
                                  sga[b]).wait()

            def scale_body(t16, _):
                wv16 = wbuf[b][pl.ds(t16 * 16, 16)]
                for lane in range(16):
                    t = t16 * 16 + lane
                    wv = lax.broadcast(wv16[lane], (16,))
                    for g in range(_H // 16):
                        rows[b][t, pl.ds(g * 16, 16)] = (
                            rows[b][t, pl.ds(g * 16, 16)] * wv)
                return 0
            lax.fori_loop(0, _C // 16, scale_body, 0)
            pltpu.async_copy(rows[b], acc_sh.at[eidx[b].at[1]], ssc[b],
                             add=True)

        pltpu.sync_copy(src_hbm.at[pl.ds(ebase0, _C)], ei0.at[0])
        pltpu.sync_copy(dst_hbm.at[pl.ds(ebase0, _C)], ei0.at[1])
        pltpu.async_copy(h_hbm.at[ei0.at[0]], r0, sga0)
        do_chunk(0, 0, True, True)
        do_chunk(1, 1, False, True)

        def main_body(k2, _):
            k = 2 + k2 * 2
            do_chunk(k, 0, False, True)
            do_chunk(k + 1, 1, False, True)
            return 0
        lax.fori_loop(0, (nch - 5) // 2, main_body, 0)

        do_chunk(nch - 3, (nch - 3) % 2, False, True)
        do_chunk(nch - 2, (nch - 2) % 2, False, True)
        do_chunk(nch - 1, (nch - 1) % 2, False, False)
        for b in ((nch - 2) % 2, (nch - 1) % 2):
            pltpu.make_async_copy(
                rows[b], acc_sh.at[eidx[b].at[1]], ssc[b]).wait()
            pltpu.make_async_copy(
                wbuf[b], den_sh.at[eidx[b].at[1]], ssd[b]).wait()

        plsc.subcore_barrier()
        pltpu.sync_copy(acc_sh.at[pl.ds(rbase, _DPT)],
                        acc_hbm.at[cid, pl.ds(rbase, _DPT)])
        pltpu.sync_copy(den_sh.at[pl.ds(rbase, _DPT)],
                        den_hbm.at[cid, pl.ds(rbase, _DPT)])

    return edge_kernel


_edge_call = _build_edge_kernel()


def kernel(x, edge_index, W_in, b_in, W_gat, att_src, att_dst, b_gat,
           W_ih, W_hh, W_out, b_out):
    src = edge_index[0]
    dst = edge_index[1]
    cur = _mm_bias(x, W_in, b_in)
    h_st = jnp.zeros((_N, _H), jnp.float32)
    c_st = jnp.zeros((_N, _H), jnp.float32)
    hops = W_gat.shape[0]
    for i in range(hops):
        av = jnp.zeros((_H, 8), jnp.float32)
        av = av.at[:, 0].set(att_src[i]).at[:, 1].set(att_dst[i])
        h, a2 = _stage_a(cur, W_gat[i], av)
        acc, den = _edge_call(h, a2[:, 0], a2[:, 1], src, dst)
        h_st, c_st = _stage_c(acc, den[:, :, None], b_gat[i],
                              W_ih[i].T, W_hh[i].T, h_st, c_st)
        cur = h_st
    return _mm_bias(cur, W_out, b_out)


# speedup vs baseline: 47.9285x; 1.4522x over previous
import functools

import jax
import jax.numpy as jnp
from jax import lax
from jax.experimental import pallas as pl
from jax.experimental.pallas import tpu as pltpu
from jax.experimental.pallas import tpu_sc as plsc

_N = 10000
_E = 320000
_H = 128
_NPAD = 10240
_NC, _NS = 2, 16
_T = _E // (_NC * _NS)
_C = 80
_DPT = _NPAD // _NS


def _mm_bias_body(x_ref, w_ref, b_ref, o_ref):
    o_ref[...] = jnp.dot(x_ref[...], w_ref[...],
                         preferred_element_type=jnp.float32) + b_ref[...]


def _mm_bias(x, w, b, blk=2000):
    n, _ = x.shape
    ko = w.shape[1]
    return pl.pallas_call(
        _mm_bias_body,
        grid=(n // blk,),
        in_specs=[pl.BlockSpec((blk, x.shape[1]), lambda i: (i, 0)),
                  pl.BlockSpec(w.shape, lambda i: (0, 0)),
                  pl.BlockSpec((1, ko), lambda i: (0, 0))],
        out_specs=pl.BlockSpec((blk, ko), lambda i: (i, 0)),
        out_shape=jax.ShapeDtypeStruct((n, ko), jnp.float32),
    )(x, w, b.reshape(1, -1))


def _stage_a_body(cur_ref, w_ref, av_ref, h_ref, a2_ref):
    h = jnp.dot(cur_ref[...], w_ref[...], preferred_element_type=jnp.float32)
    h_ref[...] = h
    a2_ref[...] = jnp.dot(h, av_ref[...], preferred_element_type=jnp.float32)


def _stage_a(cur, w, av, blk=2000):
    return pl.pallas_call(
        _stage_a_body,
        grid=(_N // blk,),
        in_specs=[pl.BlockSpec((blk, _H), lambda i: (i, 0)),
                  pl.BlockSpec((_H, _H), lambda i: (0, 0)),
                  pl.BlockSpec((_H, 8), lambda i: (0, 0))],
        out_specs=[pl.BlockSpec((blk, _H), lambda i: (i, 0)),
                   pl.BlockSpec((blk, 8), lambda i: (i, 0))],
        out_shape=[jax.ShapeDtypeStruct((_N, _H), jnp.float32),
                   jax.ShapeDtypeStruct((_N, 8), jnp.float32)],
    )(cur, w, av)


def _stage_c_body(acc_ref, den_ref, bg_ref, wih_ref, whh_ref, h_ref, c_ref,
                  ho_ref, co_ref):
    den = den_ref[0] + den_ref[1]
    acc = acc_ref[0] + acc_ref[1]
    bx = jnp.tanh(acc / (den + 1e-16) + bg_ref[...])
    gates = (jnp.dot(bx, wih_ref[...], preferred_element_type=jnp.float32)
             + jnp.dot(h_ref[...], whh_ref[...],
                       preferred_element_type=jnp.float32))
    gi = gates[:, 0 * _H:1 * _H]
    gf = gates[:, 1 * _H:2 * _H]
    gg = gates[:, 2 * _H:3 * _H]
    go = gates[:, 3 * _H:4 * _H]
    c_new = jax.nn.sigmoid(gf) * c_ref[...] + jax.nn.sigmoid(gi) * jnp.tanh(gg)
    co_ref[...] = c_new
    ho_ref[...] = jax.nn.sigmoid(go) * jnp.tanh(c_new)


def _stage_c(acc, den, bg, wihT, whhT, h_st, c_st, blk=2000):
    return pl.pallas_call(
        _stage_c_body,
        grid=(_N // blk,),
        in_specs=[pl.BlockSpec((2, blk, _H), lambda i: (0, i, 0)),
                  pl.BlockSpec((2, blk, 1), lambda i: (0, i, 0)),
                  pl.BlockSpec((1, _H), lambda i: (0, 0)),
                  pl.BlockSpec((_H, 4 * _H), lambda i: (0, 0)),
                  pl.BlockSpec((_H, 4 * _H), lambda i: (0, 0)),
                  pl.BlockSpec((blk, _H), lambda i: (i, 0)),
                  pl.BlockSpec((blk, _H), lambda i: (i, 0))],
        out_specs=[pl.BlockSpec((blk, _H), lambda i: (i, 0)),
                   pl.BlockSpec((blk, _H), lambda i: (i, 0))],
        out_shape=[jax.ShapeDtypeStruct((_N, _H), jnp.float32),
                   jax.ShapeDtypeStruct((_N, _H), jnp.float32)],
    )(acc, den, bg.reshape(1, -1), wihT, whhT, h_st, c_st)


def _build_edge_kernel():
    mesh = plsc.VectorSubcoreMesh(core_axis_name="c", subcore_axis_name="s")

    @functools.partial(
        pl.kernel,
        out_type=[jax.ShapeDtypeStruct((_NC, _NPAD, _H), jnp.float32),
                  jax.ShapeDtypeStruct((_NC, _NPAD), jnp.float32)],
        mesh=mesh,
        compiler_params=pltpu.CompilerParams(needs_layout_passes=False),
        scratch_types=[
            pltpu.VMEM((_N,), jnp.float32),
            pltpu.VMEM((_N,), jnp.float32),
            pltpu.VMEM((2, _C), jnp.int32),
            pltpu.VMEM((2, _C), jnp.int32),
            pltpu.VMEM((2, _C), jnp.int32),
            pltpu.VMEM((2, _C), jnp.int32),
            pltpu.VMEM((_C,), jnp.float32),
            pltpu.VMEM((_C,), jnp.float32),
            pltpu.VMEM((_C, _H), jnp.float32),
            pltpu.VMEM((_C, _H), jnp.float32),
            pltpu.VMEM((_DPT,), jnp.float32),
            pltpu.VMEM_SHARED((_NPAD, _H), jnp.float32),
            pltpu.VMEM_SHARED((_NPAD,), jnp.float32),
            pltpu.SemaphoreType.DMA,
            pltpu.SemaphoreType.DMA,
            pltpu.SemaphoreType.DMA,
            pltpu.SemaphoreType.DMA,
            pltpu.SemaphoreType.DMA,
            pltpu.SemaphoreType.DMA,
            pltpu.SemaphoreType.DMA,
            pltpu.SemaphoreType.DMA,
            pltpu.SemaphoreType.DMA,
            pltpu.SemaphoreType.DMA,
        ],
    )
    def edge_kernel(h_hbm, asrc_hbm, adst_hbm, src_hbm, dst_hbm,
                    acc_hbm, den_hbm,
                    asrc_l, adst_l, ei0, ei1, ei2, ei3, w0, w1, r0, r1, zbuf,
                    acc_sh, den_sh, sga0, sga1, ssd0, ssd1, ssc0, ssc1,
                    si0, si1, si2, si3):
        cid = lax.axis_index("c")
        sid = lax.axis_index("s")
        wid = cid * _NS + sid
        eidx = (ei0, ei1, ei2, ei3)
        wbuf = (w0, w1)
        rows = (r0, r1)
        sga = (sga0, sga1)
        ssd = (ssd0, ssd1)
        ssc = (ssc0, ssc1)
        si = (si0, si1, si2, si3)
        nch = _T // _C

        pltpu.sync_copy(asrc_hbm, asrc_l)
        pltpu.sync_copy(adst_hbm, adst_l)

        def mx_body(i, carry):
            ms, md = carry
            return (jnp.maximum(ms, asrc_l[pl.ds(i * 16, 16)]),
                    jnp.maximum(md, adst_l[pl.ds(i * 16, 16)]))
        neg = jnp.full((16,), -3.0e38, jnp.float32)
        ms, md = lax.fori_loop(0, _N // 16, mx_body, (neg, neg))
        lanes = lax.iota(jnp.int32, 16)
        for sft in (8, 4, 2, 1):
            perm = jnp.bitwise_xor(lanes, sft)
            w0[pl.ds(0, 16)] = ms
            ms = jnp.maximum(ms, plsc.load_gather(w0, [perm]))
            w0[pl.ds(0, 16)] = md
            md = jnp.maximum(md, plsc.load_gather(w0, [perm]))
        m = ms + md
        big_m = jnp.where(m >= 0.0, m, 0.2 * m)

        zrow = jnp.zeros((16,), jnp.float32)

        def zr_body(j, _):
            for g in range(_H // 16):
                r0[j, pl.ds(g * 16, 16)] = zrow
            return 0
        lax.fori_loop(0, _C, zr_body, 0)

        def zd_body(j, _):
            zbuf[pl.ds(j * 16, 16)] = zrow
            return 0
        lax.fori_loop(0, _DPT // 16, zd_body, 0)

        rbase = pl.multiple_of(sid * _DPT, 8)
        for z in range(_DPT // _C):
            pltpu.sync_copy(r0, acc_sh.at[pl.ds(rbase + z * _C, _C)])
        pltpu.sync_copy(zbuf, den_sh.at[pl.ds(rbase, _DPT)])
        plsc.subcore_barrier()

        ebase0 = wid * _T

        def issue_idx(k, q):
            nb = ebase0 + k * _C
            pltpu.async_copy(src_hbm.at[pl.ds(nb, _C)], eidx[q].at[0], si[q])
            pltpu.async_copy(dst_hbm.at[pl.ds(nb, _C)], eidx[q].at[1], si[q])

        def wait_idx(k, q):
            nb = ebase0 + k * _C
            pltpu.make_async_copy(src_hbm.at[pl.ds(nb, _C)], eidx[q].at[0],
                                  si[q]).wait()
            pltpu.make_async_copy(dst_hbm.at[pl.ds(nb, _C)], eidx[q].at[1],
                                  si[q]).wait()

        def do_chunk(k, b, q, first, last, pf2=True):
            o = 1 - b
            qn = (q + 1) % 4
            if not last:
                if pf2:
                    issue_idx(k + 2, (q + 2) % 4)
                if not first:
                    pltpu.make_async_copy(
                        rows[o], acc_sh.at[eidx[qn].at[1]], ssc[o]).wait()
                    pltpu.make_async_copy(
                        wbuf[o], den_sh.at[eidx[qn].at[1]], ssd[o]).wait()
                wait_idx(k + 1, qn)
                pltpu.async_copy(h_hbm.at[eidx[qn].at[0]], rows[o], sga[o])
            for j in range(_C // 16):
                sv = eidx[q][0, pl.ds(j * 16, 16)]
                dv = eidx[q][1, pl.ds(j * 16, 16)]
                e = (plsc.load_gather(asrc_l, [sv])
                     + plsc.load_gather(adst_l, [dv]))
                e = jnp.where(e >= 0.0, e, 0.2 * e)
                wbuf[b][pl.ds(j * 16, 16)] = jnp.exp(e - big_m)
            pltpu.async_copy(wbuf[b], den_sh.at[eidx[q].at[1]], ssd[b],
                             add=True)
            pltpu.make_async_copy(h_hbm.at[eidx[q].at[0]], rows[b],
                                  sga[b]).wait()

            def scale_body(t16, _):
                wv16 = wbuf[b][pl.ds(t16 * 16, 16)]
                for lane in range(16):
                    t = t16 * 16 + lane
                    wv = lax.broadcast(wv16[lane], (16,))
                    for g in range(_H // 16):
                        rows[b][t, pl.ds(g * 16, 16)] = (
                            rows[b][t, pl.ds(g * 16, 16)] * wv)
                return 0
            lax.fori_loop(0, _C // 16, scale_body, 0)
            pltpu.async_copy(rows[b], acc_sh.at[eidx[q].at[1]], ssc[b],
                             add=True)

        issue_idx(0, 0)
        issue_idx(1, 1)
        wait_idx(0, 0)
        pltpu.async_copy(h_hbm.at[ei0.at[0]], r0, sga0)
        do_chunk(0, 0, 0, True, False)
        do_chunk(1, 1, 1, False, False)

        def main_body(k4, _):
            k = 2 + k4 * 4
            for lane in range(4):
                do_chunk(k + lane, (2 + lane) % 2, (2 + lane) % 4,
                         False, False)
            return 0
        lax.fori_loop(0, (nch - 5) // 4, main_body, 0)

        do_chunk(nch - 3, (nch - 3) % 2, (nch - 3) % 4, False, False)
        do_chunk(nch - 2, (nch - 2) % 2, (nch - 2) % 4, False, False,
                 pf2=False)
        do_chunk(nch - 1, (nch - 1) % 2, (nch - 1) % 4, False, True)
        for k in (nch - 2, nch - 1):
            b = k % 2
            q = k % 4
            pltpu.make_async_copy(
                rows[b], acc_sh.at[eidx[q].at[1]], ssc[b]).wait()
            pltpu.make_async_copy(
                wbuf[b], den_sh.at[eidx[q].at[1]], ssd[b]).wait()

        plsc.subcore_barrier()
        pltpu.sync_copy(acc_sh.at[pl.ds(rbase, _DPT)],
                        acc_hbm.at[cid, pl.ds(rbase, _DPT)])
        pltpu.sync_copy(den_sh.at[pl.ds(rbase, _DPT)],
                        den_hbm.at[cid, pl.ds(rbase, _DPT)])

    return edge_kernel


_edge_call = _build_edge_kernel()


def kernel(x, edge_index, W_in, b_in, W_gat, att_src, att_dst, b_gat,
           W_ih, W_hh, W_out, b_out):
    src = edge_index[0]
    dst = edge_index[1]
    cur = _mm_bias(x, W_in, b_in)
    h_st = jnp.zeros((_N, _H), jnp.float32)
    c_st = jnp.zeros((_N, _H), jnp.float32)
    hops = W_gat.shape[0]
    for i in range(hops):
        av = jnp.zeros((_H, 8), jnp.float32)
        av = av.at[:, 0].set(att_src[i]).at[:, 1].set(att_dst[i])
        h, a2 = _stage_a(cur, W_gat[i], av)
        acc, den = _edge_call(h, a2[:, 0], a2[:, 1], src, dst)
        h_st, c_st = _stage_c(acc, den[:, :, None], b_gat[i],
                              W_ih[i].T, W_hh[i].T, h_st, c_st)
        cur = h_st
    return _mm_bias(cur, W_out, b_out)
